# baseline (speedup 1.0000x reference)
import functools

import jax
import jax.numpy as jnp
from jax import lax
from jax.experimental import pallas as pl
from jax.experimental.pallas import tpu as pltpu
from jax.experimental.pallas import tpu_sc as plsc

N = 10000
E = 160000
D = 5
EP = 163840
NT = 16
EPT = EP // NT
NB = EPT // 128
NP = 10240
ROWB = 1000
NR = N // ROWB

_mesh = plsc.VectorSubcoreMesh(core_axis_name="c", subcore_axis_name="s")


@functools.partial(
    pl.kernel,
    mesh=_mesh,
    out_type=jax.ShapeDtypeStruct((D, NP), jnp.float32),
    scratch_types=[
        pltpu.VMEM((128, 80), jnp.int32),
        pltpu.VMEM((128, 80), jnp.float32),
        pltpu.VMEM((640,), jnp.float32),
        pltpu.VMEM_SHARED((NP,), jnp.float32),
    ],
)
def _sc_deg(col_hbm, ew_hbm, deg_hbm, col_s, ew_s, ones_v, sdeg):
    cid = lax.axis_index("c")
    tid = lax.axis_index("s")

    def fill_ones(i, _):
        ones_v[pl.ds(i * 16, 16)] = jnp.full((16,), 1.0, jnp.float32)
        return 0

    lax.fori_loop(0, 40, fill_ones, 0)

    def dim_round(r, _):
        d = cid + 2 * r

        @pl.when(d < D)
        def _():
            pltpu.sync_copy(col_hbm.at[d, tid], col_s)
            pltpu.sync_copy(ew_hbm.at[d, tid], ew_s)
            pltpu.sync_copy(ones_v, sdeg.at[pl.ds(tid * 640, 640)])
            plsc.subcore_barrier()

            def batch(b, _):
                pltpu.sync_copy(ew_s.at[b], sdeg.at[col_s.at[b]], add=True)
                return 0

            lax.fori_loop(0, 128, batch, 0)
            plsc.subcore_barrier()
            pltpu.sync_copy(sdeg.at[pl.ds(tid * 640, 640)],
                            deg_hbm.at[d, pl.ds(tid * 640, 640)])
            plsc.subcore_barrier()

        return 0

    lax.fori_loop(0, 3, dim_round, 0)


@functools.partial(
    pl.kernel,
    mesh=_mesh,
    out_type=jax.ShapeDtypeStruct((4, NP, 128), jnp.float32),
    scratch_types=[
        pltpu.VMEM((8, 80), jnp.int32),
        pltpu.VMEM((8, 80), jnp.int32),
        pltpu.VMEM((8, 80), jnp.int32),
        pltpu.VMEM((8, 80), jnp.float32),
        pltpu.VMEM((80, 128), jnp.float32),
        pltpu.VMEM((80, 128), jnp.float32),
        pltpu.VMEM((80, 128), jnp.float32),
        pltpu.VMEM((80, 128), jnp.float32),
        pltpu.VMEM((8, 128), jnp.float32),
        pltpu.VMEM_SHARED((NP, 128), jnp.float32),
        pltpu.SemaphoreType.DMA,
        pltpu.SemaphoreType.DMA,
        pltpu.SemaphoreType.DMA,
        pltpu.SemaphoreType.DMA,
    ],
)
def _sc_prop(hp_hbm, row_hbm, col_hbm, ew_hbm, acc_hbm,
             row_s, row_off, col_s, ew_s, gbufa, gbufb, sbufa, sbufb, zbuf,
             acc_s, ga, gb, sa, sb):
    cid = lax.axis_index("c")
    tid = lax.axis_index("s")
    NBB = EPT // 80
    NG = NBB // 8

    def fill_zero(i, _):
        def fz(j, _):
            zbuf[i, pl.ds(j * 16, 16)] = jnp.zeros((16,), jnp.float32)
            return 0
        lax.fori_loop(0, 8, fz, 0)
        return 0

    lax.fori_loop(0, 16, fill_zero, 0)

    def round_body(r, _):
        rc = cid * 2 + r
        base = rc * N
        bvec = lax.broadcast_in_dim(base, (16,), ())

        def zr(k, _):
            pltpu.sync_copy(zbuf, acc_s.at[pl.ds(tid * 640 + k * 8, 8)])
            return 0

        lax.fori_loop(0, 80, zr, 0)
        plsc.subcore_barrier()

        def mul(dst, src, b):
            def mj(jj, _):
                wv = ew_s[b, pl.ds(jj * 16, 16)]
                for l in range(16):
                    w = lax.broadcast_in_dim(wv[l], (16,), ())
                    j = jj * 16 + l
                    for k in range(8):
                        sl = pl.ds(k * 16, 16)
                        dst[j, sl] = src[j, sl] * w
                return 0

            lax.fori_loop(0, 5, mj, 0)

        def group(g, _):
            pltpu.sync_copy(row_hbm.at[tid, pl.ds(g * 8, 8)], row_s)
            pltpu.sync_copy(col_hbm.at[tid, pl.ds(g * 8, 8)], col_s)
            pltpu.sync_copy(ew_hbm.at[tid, pl.ds(g * 8, 8)], ew_s)

            def off_body(b, _):
                def oj(j, _):
                    sl = pl.ds(j * 16, 16)
                    row_off[b, sl] = row_s[b, sl] + bvec
                    return 0
                lax.fori_loop(0, 5, oj, 0)
                return 0

            lax.fori_loop(0, 8, off_body, 0)

            pltpu.async_copy(hp_hbm.at[row_off.at[0]], gbufa, ga)
            pltpu.async_copy(hp_hbm.at[row_off.at[1]], gbufb, gb)

            def pair(ii, _):
                b0 = ii * 2
                b1 = b0 + 1
                pltpu.make_async_copy(
                    hp_hbm.at[row_off.at[0]], gbufa, ga).wait()

                @pl.when(ii > 0)
                def _():
                    pltpu.make_async_copy(
                        sbufa, acc_s.at[col_s.at[0]], sa).wait()

                mul(sbufa, gbufa, b0)

                @pl.when(ii < 3)
                def _():
                    pltpu.async_copy(hp_hbm.at[row_off.at[b0 + 2]],
                                     gbufa, ga)

                pltpu.async_copy(sbufa, acc_s.at[col_s.at[b0]], sa,
                                 add=True)

                pltpu.make_async_copy(
                    hp_hbm.at[row_off.at[0]], gbufb, gb).wait()

                @pl.when(ii > 0)
                def _():
                    pltpu.make_async_copy(
                        sbufb, acc_s.at[col_s.at[0]], sb).wait()

                mul(sbufb, gbufb, b1)

                @pl.when(ii < 3)
                def _():
                    pltpu.async_copy(hp_hbm.at[row_off.at[b1 + 2]],
                                     gbufb, gb)

                pltpu.async_copy(sbufb, acc_s.at[col_s.at[b1]], sb,
                                 add=True)
                return 0

            lax.fori_loop(0, 4, pair, 0)
            pltpu.make_async_copy(sbufa, acc_s.at[col_s.at[0]], sa).wait()
            pltpu.make_async_copy(sbufb, acc_s.at[col_s.at[0]], sb).wait()
            return 0

        lax.fori_loop(0, NG, group, 0)
        plsc.subcore_barrier()
        pltpu.sync_copy(acc_s.at[pl.ds(tid * 640, 640)],
                        acc_hbm.at[rc, pl.ds(tid * 640, 640)])
        plsc.subcore_barrier()
        return 0

    lax.fori_loop(0, 2, round_body, 0)


def _k1_body(x_ref, w_ref, b_ref, o_ref):
    o_ref[...] = jnp.dot(x_ref[...], w_ref[...],
                         preferred_element_type=jnp.float32) + b_ref[...]


def _k1(x, w, b2):
    cin = x.shape[1]
    return pl.pallas_call(
        _k1_body,
        grid=(NR,),
        in_specs=[
            pl.BlockSpec((ROWB, cin), lambda r: (r, 0)),
            pl.BlockSpec((cin, 512), lambda r: (0, 0)),
            pl.BlockSpec((1, 512), lambda r: (0, 0)),
        ],
        out_specs=pl.BlockSpec((ROWB, 512), lambda r: (r, 0)),
        out_shape=jax.ShapeDtypeStruct((N, 512), jnp.float32),
    )(x, w, b2)


def _k2_body(lin_ref, cw_ref, dinv_ref, o_ref):
    h = jnp.dot(lin_ref[...], cw_ref[...], preferred_element_type=jnp.float32)
    o_ref[0] = h * dinv_ref[...]


def _k2(lin, cw, dinv2):
    return pl.pallas_call(
        _k2_body,
        grid=(NR, 4),
        in_specs=[
            pl.BlockSpec((ROWB, 512), lambda r, c: (r, 0)),
            pl.BlockSpec((512, 128), lambda r, c: (0, c)),
            pl.BlockSpec((ROWB, 1), lambda r, c: (r, 0)),
        ],
        out_specs=pl.BlockSpec((1, ROWB, 128), lambda r, c: (c, r, 0)),
        out_shape=jax.ShapeDtypeStruct((4, N, 128), jnp.float32),
    )(lin, cw, dinv2)


def _k3_body(lin_ref, hp_ref, acc_ref, dinv_ref, cb_ref, o_ref):
    g = dinv_ref[...] * (acc_ref[0] + hp_ref[0]) + cb_ref[...]
    o_ref[...] = jnp.maximum(lin_ref[...] + g, 0.0)


def _k3(lin, hp, acc, dinv2, cb2):
    return pl.pallas_call(
        _k3_body,
        grid=(NR, 4),
        in_specs=[
            pl.BlockSpec((ROWB, 128), lambda r, c: (r, c)),
            pl.BlockSpec((1, ROWB, 128), lambda r, c: (c, r, 0)),
            pl.BlockSpec((1, ROWB, 128), lambda r, c: (c, r, 0)),
            pl.BlockSpec((ROWB, 1), lambda r, c: (r, 0)),
            pl.BlockSpec((1, 128), lambda r, c: (0, c)),
        ],
        out_specs=pl.BlockSpec((ROWB, 128), lambda r, c: (r, c)),
        out_shape=jax.ShapeDtypeStruct((N, 512), jnp.float32),
    )(lin, hp, acc, dinv2, cb2)


def _ksum_body(x0, x1, x2, x3, x4, o_ref):
    o_ref[...] = (x0[...] + x1[...] + x2[...] + x3[...] + x4[...]
                  + jnp.float32(1e-8))


def _ksum(xs):
    spec = pl.BlockSpec((ROWB, 512), lambda r: (r, 0))
    return pl.pallas_call(
        _ksum_body,
        grid=(NR,),
        in_specs=[spec] * 5,
        out_specs=spec,
        out_shape=jax.ShapeDtypeStruct((N, 512), jnp.float32),
    )(*xs)


def _rsqrt_body(deg_ref, o_ref):
    o_ref[...] = lax.rsqrt(deg_ref[...])


def _rsqrt(deg):
    return pl.pallas_call(
        _rsqrt_body,
        out_shape=jax.ShapeDtypeStruct((D, NP), jnp.float32),
    )(deg)


def kernel(dim1, dim2, dim3, dim4, dim5, edge_indices, edge_weights, non_zero,
           lin_w1, lin_b1, conv_w1, conv_b1, lin_w2, lin_b2, conv_w2, conv_b2,
           lin_w3, lin_b3, conv_w3, conv_b3, lin_w4, lin_b4, conv_w4, conv_b4,
           lin_w5, lin_b5, conv_w5, conv_b5, lin_w6, lin_b6, conv_w6, conv_b6):
    xs = [dim1, dim2, dim3, dim4, dim5]
    lws = [lin_w1, lin_w2, lin_w3, lin_w4, lin_w5, lin_w6]
    lbs = [lin_b1, lin_b2, lin_b3, lin_b4, lin_b5, lin_b6]
    cws = [conv_w1, conv_w2, conv_w3, conv_w4, conv_w5, conv_w6]
    cbs = [conv_b1, conv_b2, conv_b3, conv_b4, conv_b5, conv_b6]

    row = edge_indices[:, 0, :]
    col = edge_indices[:, 1, :]
    pad_idx = jnp.broadcast_to(
        (jnp.arange(E, EP, dtype=jnp.int32) % N)[None, :], (D, EP - E))
    row_p = jnp.concatenate([row, pad_idx], axis=1).reshape(D, NT, 128, 80)
    col_p = jnp.concatenate([col, pad_idx], axis=1).reshape(D, NT, 128, 80)
    ew_p = jnp.concatenate(
        [edge_weights, jnp.zeros((D, EP - E), jnp.float32)],
        axis=1).reshape(D, NT, 128, 80)

    deg = _sc_deg(col_p, ew_p)
    dinv = _rsqrt(deg)
    dinv2 = [dinv[d, :N].reshape(N, 1) for d in range(D)]

    for l in range(6):
        b2 = lbs[l].reshape(1, 512)
        new_xs = []
        for d in range(D):
            lin = _k1(xs[d], lws[l], b2)
            hp = _k2(lin, cws[l][d], dinv2[d])
            acc = _sc_prop(hp.reshape(4 * N, 128), row_p[d], col_p[d],
                           ew_p[d])
            new_xs.append(_k3(lin, hp, acc, dinv2[d],
                              cbs[l][d].reshape(1, 512)))
        xs = new_xs
    return _ksum(xs)


# speedup vs baseline: 1.3420x; 1.3420x over previous
import functools

import jax
import jax.numpy as jnp
from jax import lax
from jax.experimental import pallas as pl
from jax.experimental.pallas import tpu as pltpu
from jax.experimental.pallas import tpu_sc as plsc

N = 10000
E = 160000
D = 5
EP = 163840
NT = 16
EPT = EP // NT
NB = EPT // 128
NP = 10240
ROWB = 1000
NR = N // ROWB

_mesh = plsc.VectorSubcoreMesh(core_axis_name="c", subcore_axis_name="s")


@functools.partial(
    pl.kernel,
    mesh=_mesh,
    out_type=jax.ShapeDtypeStruct((D, NP), jnp.float32),
    scratch_types=[
        pltpu.VMEM((NB, 128), jnp.int32),
        pltpu.VMEM((NB, 128), jnp.float32),
        pltpu.VMEM((640,), jnp.float32),
        pltpu.VMEM_SHARED((NP,), jnp.float32),
    ],
)
def _sc_deg(col_hbm, ew_hbm, deg_hbm, col_s, ew_s, ones_v, sdeg):
    cid = lax.axis_index("c")
    tid = lax.axis_index("s")

    def fill_ones(i, _):
        ones_v[pl.ds(i * 16, 16)] = jnp.full((16,), 1.0, jnp.float32)
        return 0

    lax.fori_loop(0, 40, fill_ones, 0)

    def dim_round(r, _):
        d = cid + 2 * r

        @pl.when(d < D)
        def _():
            pltpu.sync_copy(col_hbm.at[d, tid], col_s)
            pltpu.sync_copy(ew_hbm.at[d, tid], ew_s)
            pltpu.sync_copy(ones_v, sdeg.at[pl.ds(tid * 640, 640)])
            plsc.subcore_barrier()

            def batch(b, _):
                pltpu.sync_copy(ew_s.at[b], sdeg.at[col_s.at[b]], add=True)
                return 0

            lax.fori_loop(0, NB, batch, 0)
            plsc.subcore_barrier()
            pltpu.sync_copy(sdeg.at[pl.ds(tid * 640, 640)],
                            deg_hbm.at[d, pl.ds(tid * 640, 640)])
            plsc.subcore_barrier()

        return 0

    lax.fori_loop(0, 3, dim_round, 0)


@functools.partial(
    pl.kernel,
    mesh=_mesh,
    out_type=jax.ShapeDtypeStruct((4, NP, 128), jnp.float32),
    scratch_types=[
        pltpu.VMEM((16, 128), jnp.int32),
        pltpu.VMEM((16, 128), jnp.int32),
        pltpu.VMEM((16, 128), jnp.int32),
        pltpu.VMEM((16, 128), jnp.float32),
        pltpu.VMEM((128, 128), jnp.float32),
        pltpu.VMEM((128, 128), jnp.float32),
        pltpu.VMEM((64, 128), jnp.float32),
        pltpu.VMEM_SHARED((NP, 128), jnp.float32),
        pltpu.SemaphoreType.DMA,
        pltpu.SemaphoreType.DMA,
        pltpu.SemaphoreType.DMA,
        pltpu.SemaphoreType.DMA,
    ],
)
def _sc_prop(hp_hbm, row_hbm, col_hbm, ew_hbm, acc_hbm,
             row_s, row_off, col_s, ew_s, gbufa, gbufb, zbuf, acc_s,
             ga, gb, sa, sb):
    cid = lax.axis_index("c")
    tid = lax.axis_index("s")
    NG = NB // 16

    def fill_zero(i, _):
        def fz(j, _):
            zbuf[i, pl.ds(j * 16, 16)] = jnp.zeros((16,), jnp.float32)
            return 0
        lax.fori_loop(0, 8, fz, 0)
        return 0

    lax.fori_loop(0, 64, fill_zero, 0)

    def round_body(r, _):
        rc = cid * 2 + r
        base = rc * N
        bvec = lax.broadcast_in_dim(base, (16,), ())

        def zr(k, _):
            pltpu.sync_copy(zbuf, acc_s.at[pl.ds(tid * 640 + k * 64, 64)])
            return 0

        lax.fori_loop(0, 10, zr, 0)
        plsc.subcore_barrier()

        def mul(buf, b):
            def mj(jj, _):
                wv = ew_s[b, pl.ds(jj * 16, 16)]
                for l in range(16):
                    w = lax.broadcast_in_dim(wv[l], (16,), ())
                    j = jj * 16 + l
                    for k in range(8):
                        sl = pl.ds(k * 16, 16)
                        buf[j, sl] = buf[j, sl] * w
                return 0

            lax.fori_loop(0, 8, mj, 0)

        def group(g, _):
            pltpu.sync_copy(row_hbm.at[tid, pl.ds(g * 16, 16)], row_s)
            pltpu.sync_copy(col_hbm.at[tid, pl.ds(g * 16, 16)], col_s)
            pltpu.sync_copy(ew_hbm.at[tid, pl.ds(g * 16, 16)], ew_s)

            def off_body(b, _):
                def oj(j, _):
                    sl = pl.ds(j * 16, 16)
                    row_off[b, sl] = row_s[b, sl] + bvec
                    return 0
                lax.fori_loop(0, 8, oj, 0)
                return 0

            lax.fori_loop(0, 16, off_body, 0)

            pltpu.async_copy(hp_hbm.at[row_off.at[0]], gbufa, ga)

            def pair(ii, _):
                b0 = ii * 2
                b1 = b0 + 1
                pltpu.make_async_copy(
                    hp_hbm.at[row_off.at[0]], gbufa, ga).wait()

                @pl.when(ii > 0)
                def _():
                    pltpu.make_async_copy(
                        gbufb, acc_s.at[col_s.at[0]], sb).wait()

                pltpu.async_copy(hp_hbm.at[row_off.at[b1]], gbufb, gb)
                mul(gbufa, b0)
                pltpu.async_copy(gbufa, acc_s.at[col_s.at[b0]], sa,
                                 add=True)

                pltpu.make_async_copy(
                    hp_hbm.at[row_off.at[0]], gbufb, gb).wait()

                @pl.when(ii < 7)
                def _():
                    pltpu.make_async_copy(
                        gbufa, acc_s.at[col_s.at[0]], sa).wait()
                    pltpu.async_copy(hp_hbm.at[row_off.at[b0 + 2]],
                                     gbufa, ga)

                mul(gbufb, b1)
                pltpu.async_copy(gbufb, acc_s.at[col_s.at[b1]], sb,
                                 add=True)
                return 0

            lax.fori_loop(0, 8, pair, 0)
            pltpu.make_async_copy(gbufa, acc_s.at[col_s.at[0]], sa).wait()
            pltpu.make_async_copy(gbufb, acc_s.at[col_s.at[0]], sb).wait()
            return 0

        lax.fori_loop(0, NG, group, 0)
        plsc.subcore_barrier()
        pltpu.sync_copy(acc_s.at[pl.ds(tid * 640, 640)],
                        acc_hbm.at[rc, pl.ds(tid * 640, 640)])
        plsc.subcore_barrier()
        return 0

    lax.fori_loop(0, 2, round_body, 0)


def _k1_body(x_ref, w_ref, b_ref, o_ref):
    o_ref[...] = jnp.dot(x_ref[...], w_ref[...],
                         preferred_element_type=jnp.float32) + b_ref[...]


def _k1(x, w, b2):
    cin = x.shape[1]
    return pl.pallas_call(
        _k1_body,
        grid=(NR,),
        in_specs=[
            pl.BlockSpec((ROWB, cin), lambda r: (r, 0)),
            pl.BlockSpec((cin, 512), lambda r: (0, 0)),
            pl.BlockSpec((1, 512), lambda r: (0, 0)),
        ],
        out_specs=pl.BlockSpec((ROWB, 512), lambda r: (r, 0)),
        out_shape=jax.ShapeDtypeStruct((N, 512), jnp.float32),
    )(x, w, b2)


def _k2_body(lin_ref, cw_ref, dinv_ref, o_ref):
    h = jnp.dot(lin_ref[...], cw_ref[...], preferred_element_type=jnp.float32)
    o_ref[0] = h * dinv_ref[...]


def _k2(lin, cw, dinv2):
    return pl.pallas_call(
        _k2_body,
        grid=(NR, 4),
        in_specs=[
            pl.BlockSpec((ROWB, 512), lambda r, c: (r, 0)),
            pl.BlockSpec((512, 128), lambda r, c: (0, c)),
            pl.BlockSpec((ROWB, 1), lambda r, c: (r, 0)),
        ],
        out_specs=pl.BlockSpec((1, ROWB, 128), lambda r, c: (c, r, 0)),
        out_shape=jax.ShapeDtypeStruct((4, N, 128), jnp.float32),
    )(lin, cw, dinv2)


def _k3_body(lin_ref, hp_ref, acc_ref, dinv_ref, cb_ref, o_ref):
    g = dinv_ref[...] * (acc_ref[0] + hp_ref[0]) + cb_ref[...]
    o_ref[...] = jnp.maximum(lin_ref[...] + g, 0.0)


def _k3(lin, hp, acc, dinv2, cb2):
    return pl.pallas_call(
        _k3_body,
        grid=(NR, 4),
        in_specs=[
            pl.BlockSpec((ROWB, 128), lambda r, c: (r, c)),
            pl.BlockSpec((1, ROWB, 128), lambda r, c: (c, r, 0)),
            pl.BlockSpec((1, ROWB, 128), lambda r, c: (c, r, 0)),
            pl.BlockSpec((ROWB, 1), lambda r, c: (r, 0)),
            pl.BlockSpec((1, 128), lambda r, c: (0, c)),
        ],
        out_specs=pl.BlockSpec((ROWB, 128), lambda r, c: (r, c)),
        out_shape=jax.ShapeDtypeStruct((N, 512), jnp.float32),
    )(lin, hp, acc, dinv2, cb2)


def _ksum_body(x0, x1, x2, x3, x4, o_ref):
    o_ref[...] = (x0[...] + x1[...] + x2[...] + x3[...] + x4[...]
                  + jnp.float32(1e-8))


def _ksum(xs):
    spec = pl.BlockSpec((ROWB, 512), lambda r: (r, 0))
    return pl.pallas_call(
        _ksum_body,
        grid=(NR,),
        in_specs=[spec] * 5,
        out_specs=spec,
        out_shape=jax.ShapeDtypeStruct((N, 512), jnp.float32),
    )(*xs)


def _rsqrt_body(deg_ref, o_ref):
    o_ref[...] = lax.rsqrt(deg_ref[...])


def _rsqrt(deg):
    return pl.pallas_call(
        _rsqrt_body,
        out_shape=jax.ShapeDtypeStruct((D, NP), jnp.float32),
    )(deg)


def kernel(dim1, dim2, dim3, dim4, dim5, edge_indices, edge_weights, non_zero,
           lin_w1, lin_b1, conv_w1, conv_b1, lin_w2, lin_b2, conv_w2, conv_b2,
           lin_w3, lin_b3, conv_w3, conv_b3, lin_w4, lin_b4, conv_w4, conv_b4,
           lin_w5, lin_b5, conv_w5, conv_b5, lin_w6, lin_b6, conv_w6, conv_b6):
    xs = [dim1, dim2, dim3, dim4, dim5]
    lws = [lin_w1, lin_w2, lin_w3, lin_w4, lin_w5, lin_w6]
    lbs = [lin_b1, lin_b2, lin_b3, lin_b4, lin_b5, lin_b6]
    cws = [conv_w1, conv_w2, conv_w3, conv_w4, conv_w5, conv_w6]
    cbs = [conv_b1, conv_b2, conv_b3, conv_b4, conv_b5, conv_b6]

    row = edge_indices[:, 0, :]
    col = edge_indices[:, 1, :]
    pad_idx = jnp.broadcast_to(
        (jnp.arange(E, EP, dtype=jnp.int32) % N)[None, :], (D, EP - E))
    row_p = jnp.concatenate([row, pad_idx], axis=1).reshape(D, NT, NB, 128)
    col_p = jnp.concatenate([col, pad_idx], axis=1).reshape(D, NT, NB, 128)
    ew_p = jnp.concatenate(
        [edge_weights, jnp.zeros((D, EP - E), jnp.float32)],
        axis=1).reshape(D, NT, NB, 128)

    deg = _sc_deg(col_p, ew_p)
    dinv = _rsqrt(deg)
    dinv2 = [dinv[d, :N].reshape(N, 1) for d in range(D)]

    for l in range(6):
        b2 = lbs[l].reshape(1, 512)
        new_xs = []
        for d in range(D):
            lin = _k1(xs[d], lws[l], b2)
            hp = _k2(lin, cws[l][d], dinv2[d])
            acc = _sc_prop(hp.reshape(4 * N, 128), row_p[d], col_p[d],
                           ew_p[d])
            new_xs.append(_k3(lin, hp, acc, dinv2[d],
                              cbs[l][d].reshape(1, 512)))
        xs = new_xs
    return _ksum(xs)
